# baseline (speedup 1.0000x reference)
import functools

import jax
import jax.numpy as jnp
import numpy as np
from jax import lax
from jax.experimental import pallas as pl
from jax.experimental.pallas import tpu as pltpu
from jax.experimental.pallas import tpu_sc as plsc

N_NODES = 10000
FDIM = 128
NC = 2
NS = 16
NW = NC * NS
NPAD = 10240
TRASH = NPAD - N_NODES
ECHUNK = 128
NCH_REAL = 2500
CPT = 80
HC = 40
TAIL = NCH_REAL - (NW - 1) * CPT
EX = CPT - 16
RPT = NPAD // NS
PADE = (EX - 4) * ECHUNK
BLK = 1024
GRID = NPAD // BLK

_mesh = plsc.VectorSubcoreMesh(core_axis_name="c", subcore_axis_name="s")
_sc_params = pltpu.CompilerParams(needs_layout_passes=False)


def _count_body(edge_hbm, extra_hbm, ones_hbm, zeros_hbm, out0, out1,
                dst_v, ones_v, sem, cnt_sh):
    c = lax.axis_index("c")
    s = lax.axis_index("s")
    wid = c * NS + s

    pltpu.sync_copy(zeros_hbm, cnt_sh.at[pl.ds(s * RPT, RPT)])
    pltpu.sync_copy(ones_hbm, ones_v)

    @pl.when(wid < NW - 1)
    def _():
        pltpu.sync_copy(edge_hbm.at[1, pl.ds(wid * CPT, CPT)], dst_v)

    @pl.when(wid == NW - 1)
    def _():
        pltpu.sync_copy(edge_hbm.at[1, pl.ds((NW - 1) * CPT, 16)],
                        dst_v.at[pl.ds(0, 16)])
        pltpu.sync_copy(extra_hbm.at[1], dst_v.at[pl.ds(16, EX)])

    plsc.subcore_barrier()

    def _step(g, carry):
        for u in range(8):
            pltpu.async_copy(ones_v, cnt_sh.at[dst_v.at[g * 8 + u]], sem,
                             add=True)
        for u in range(8):
            pltpu.make_async_copy(ones_v, cnt_sh.at[dst_v.at[g * 8 + u]],
                                  sem).wait()
        return carry

    lax.fori_loop(0, CPT // 8, _step, 0)
    plsc.subcore_barrier()

    @pl.when(c == 0)
    def _():
        pltpu.sync_copy(cnt_sh.at[pl.ds(s * RPT, RPT)],
                        out0.at[pl.ds(s * RPT, RPT)])

    @pl.when(c == 1)
    def _():
        pltpu.sync_copy(cnt_sh.at[pl.ds(s * RPT, RPT)],
                        out1.at[pl.ds(s * RPT, RPT)])


_count_call = pl.kernel(
    _count_body,
    out_type=[jax.ShapeDtypeStruct((NPAD,), jnp.float32),
              jax.ShapeDtypeStruct((NPAD,), jnp.float32)],
    mesh=_mesh,
    scratch_types=[
        pltpu.VMEM((CPT, ECHUNK), jnp.int32),
        pltpu.VMEM((ECHUNK,), jnp.float32),
        pltpu.SemaphoreType.DMA,
        pltpu.VMEM_SHARED((NPAD,), jnp.float32),
    ],
    compiler_params=_sc_params,
)


def _scatter_body(coef, *refs):
    if coef:
        (table_hbm, edge_hbm, extra_hbm, w_hbm, zeros_hbm,
         part_out, cp0_out, cp1_out,
         src_v, dst_v, rows_a, rows_b, sem_a, sem_b,
         wbuf_a, wbuf_b, sem_wa, sem_wb, sem_c, acc_sh, c_sh) = refs
    else:
        (table_hbm, edge_hbm, extra_hbm, part_out,
         src_v, dst_v, rows_a, rows_b, sem_a, sem_b, acc_sh) = refs
    c = lax.axis_index("c")
    s = lax.axis_index("s")
    wid = c * NS + s

    def _zrow(r, carry):
        for k in range(FDIM // 16):
            rows_b[r, pl.ds(k * 16, 16)] = jnp.zeros((16,), jnp.float32)
        return carry

    lax.fori_loop(0, ECHUNK, _zrow, 0)
    for t in range(RPT // ECHUNK):
        pltpu.async_copy(rows_b,
                         acc_sh.at[pl.ds(s * RPT + t * ECHUNK, ECHUNK)],
                         sem_b)
    if coef:
        pltpu.sync_copy(zeros_hbm, c_sh.at[pl.ds(s * RPT, RPT)])

    def _gather(j, rbuf, rsem, wbuf, wsem, climit):
        pltpu.async_copy(table_hbm.at[src_v.at[j]], rbuf, rsem)
        if coef:
            @pl.when(j < climit)
            def _():
                pltpu.async_copy(w_hbm.at[dst_v.at[j]], wbuf, wsem)

    def _drain_scatter(j, rbuf, rsem, wbuf, wsem, climit):
        pltpu.make_async_copy(table_hbm.at[src_v.at[j]], rbuf, rsem).wait()
        if coef:
            @pl.when(j < climit)
            def _():
                pltpu.make_async_copy(w_hbm.at[dst_v.at[j]], wbuf,
                                      wsem).wait()
                pltpu.async_copy(wbuf, c_sh.at[src_v.at[j]], sem_c, add=True)
        pltpu.sync_copy(rbuf, acc_sh.at[dst_v.at[j]], add=True)
        if coef:
            @pl.when(j < climit)
            def _():
                pltpu.make_async_copy(wbuf, c_sh.at[src_v.at[j]],
                                      sem_c).wait()

    def _pipeline(climit):
        _gather(0, rows_a, sem_a, wbuf_a if coef else None,
                sem_wa if coef else None, climit)

        def _step(j2, carry):
            j = j2 * 2
            _gather(j + 1, rows_b, sem_b, wbuf_b if coef else None,
                    sem_wb if coef else None, climit)
            _drain_scatter(j, rows_a, sem_a, wbuf_a if coef else None,
                           sem_wa if coef else None, climit)

            @pl.when(j + 2 < HC)
            def _():
                _gather(j + 2, rows_a, sem_a, wbuf_a if coef else None,
                        sem_wa if coef else None, climit)

            _drain_scatter(j + 1, rows_b, sem_b, wbuf_b if coef else None,
                           sem_wb if coef else None, climit)
            return carry

        lax.fori_loop(0, HC // 2, _step, 0)

    last = wid == NW - 1
    @pl.when(jnp.logical_not(last))
    def _():
        pltpu.sync_copy(edge_hbm.at[0, pl.ds(wid * CPT, HC)], src_v)
        pltpu.sync_copy(edge_hbm.at[1, pl.ds(wid * CPT, HC)], dst_v)

    @pl.when(last)
    def _():
        pltpu.sync_copy(edge_hbm.at[0, pl.ds((NW - 1) * CPT, 16)],
                        src_v.at[pl.ds(0, 16)])
        pltpu.sync_copy(edge_hbm.at[1, pl.ds((NW - 1) * CPT, 16)],
                        dst_v.at[pl.ds(0, 16)])
        pltpu.sync_copy(extra_hbm.at[0, pl.ds(0, HC - 16)],
                        src_v.at[pl.ds(16, HC - 16)])
        pltpu.sync_copy(extra_hbm.at[1, pl.ds(0, HC - 16)],
                        dst_v.at[pl.ds(16, HC - 16)])

    for t in range(RPT // ECHUNK):
        pltpu.make_async_copy(
            rows_b, acc_sh.at[pl.ds(s * RPT + t * ECHUNK, ECHUNK)],
            sem_b).wait()
    plsc.subcore_barrier()
    _pipeline(jnp.where(last, 20, HC))
    @pl.when(jnp.logical_not(last))
    def _():
        pltpu.sync_copy(edge_hbm.at[0, pl.ds(wid * CPT + HC, HC)], src_v)
        pltpu.sync_copy(edge_hbm.at[1, pl.ds(wid * CPT + HC, HC)], dst_v)

    @pl.when(last)
    def _():
        pltpu.sync_copy(extra_hbm.at[0, pl.ds(HC - 16, HC)], src_v)
        pltpu.sync_copy(extra_hbm.at[1, pl.ds(HC - 16, HC)], dst_v)

    _pipeline(jnp.where(last, 0, HC))

    plsc.subcore_barrier()
    pltpu.sync_copy(acc_sh.at[pl.ds(s * RPT, RPT)],
                    part_out.at[c, pl.ds(s * RPT, RPT)])
    if coef:
        @pl.when(c == 0)
        def _():
            pltpu.sync_copy(c_sh.at[pl.ds(s * RPT, RPT)],
                            cp0_out.at[pl.ds(s * RPT, RPT)])

        @pl.when(c == 1)
        def _():
            pltpu.sync_copy(c_sh.at[pl.ds(s * RPT, RPT)],
                            cp1_out.at[pl.ds(s * RPT, RPT)])


def _make_scatter(coef):
    part_type = jax.ShapeDtypeStruct((NC, NPAD, FDIM), jnp.float32)
    out_type = [part_type]
    scratch = [
        pltpu.VMEM((HC, ECHUNK), jnp.int32),
        pltpu.VMEM((HC, ECHUNK), jnp.int32),
        pltpu.VMEM((ECHUNK, FDIM), jnp.float32),
        pltpu.VMEM((ECHUNK, FDIM), jnp.float32),
        pltpu.SemaphoreType.DMA,
        pltpu.SemaphoreType.DMA,
    ]
    if coef:
        out_type += [jax.ShapeDtypeStruct((NPAD,), jnp.float32),
                     jax.ShapeDtypeStruct((NPAD,), jnp.float32)]
        scratch += [
            pltpu.VMEM((ECHUNK,), jnp.float32),
            pltpu.VMEM((ECHUNK,), jnp.float32),
            pltpu.SemaphoreType.DMA,
            pltpu.SemaphoreType.DMA,
            pltpu.SemaphoreType.DMA,
        ]
    scratch.append(pltpu.VMEM_SHARED((NPAD, FDIM), jnp.float32))
    if coef:
        scratch.append(pltpu.VMEM_SHARED((NPAD,), jnp.float32))
    return pl.kernel(
        functools.partial(_scatter_body, coef),
        out_type=out_type if coef else part_type,
        mesh=_mesh,
        scratch_types=scratch,
        compiler_params=_sc_params,
    )


_scatter_coef_call = _make_scatter(True)
_scatter_call = _make_scatter(False)


def _mm_body(x_ref, w_ref, o_ref):
    o_ref[...] = jnp.dot(x_ref[...], w_ref[...],
                         preferred_element_type=jnp.float32)


def _mm_call(x, W):
    nblk = x.shape[0] // GRID
    return pl.pallas_call(
        _mm_body,
        grid=(GRID,),
        in_specs=[
            pl.BlockSpec((nblk, FDIM), lambda i: (i, 0)),
            pl.BlockSpec((FDIM, FDIM), lambda i: (0, 0)),
        ],
        out_specs=pl.BlockSpec((nblk, FDIM), lambda i: (i, 0)),
        out_shape=jax.ShapeDtypeStruct((NPAD, FDIM), jnp.float32),
    )(x, W)


def _k2_body(xw_ref, c0_ref, c1_ref, y_ref, dinv_ref, wcol_ref):
    cnt = c0_ref[...] + c1_ref[...]
    dinv = lax.rsqrt(cnt + 1.0)
    y_ref[...] = xw_ref[...] * jnp.reshape(dinv, (BLK, 1))
    dinv_ref[...] = dinv
    wcol_ref[...] = 1.0 / jnp.maximum(cnt, 1.0)


def _k2_call(xw, cnt0, cnt1):
    return pl.pallas_call(
        _k2_body,
        grid=(GRID,),
        in_specs=[
            pl.BlockSpec((BLK, FDIM), lambda i: (i, 0)),
            pl.BlockSpec((BLK,), lambda i: (i,)),
            pl.BlockSpec((BLK,), lambda i: (i,)),
        ],
        out_specs=[
            pl.BlockSpec((BLK, FDIM), lambda i: (i, 0)),
            pl.BlockSpec((BLK,), lambda i: (i,)),
            pl.BlockSpec((BLK,), lambda i: (i,)),
        ],
        out_shape=[
            jax.ShapeDtypeStruct((NPAD, FDIM), jnp.float32),
            jax.ShapeDtypeStruct((NPAD,), jnp.float32),
            jax.ShapeDtypeStruct((NPAD,), jnp.float32),
        ],
    )(xw, cnt0, cnt1)


def _k4_body(p_ref, y_ref, dinv_ref, b_ref, h0_ref):
    dcol = jnp.reshape(dinv_ref[...], (BLK, 1))
    t = (p_ref[0] + p_ref[1] + y_ref[...]) * dcol
    h0_ref[...] = jnp.maximum(t + b_ref[...], 0.0)


def _k4_call(parts, y, dinv_col, b_gcn2):
    return pl.pallas_call(
        _k4_body,
        grid=(GRID,),
        in_specs=[
            pl.BlockSpec((NC, BLK, FDIM), lambda i: (0, i, 0)),
            pl.BlockSpec((BLK, FDIM), lambda i: (i, 0)),
            pl.BlockSpec((BLK,), lambda i: (i,)),
            pl.BlockSpec((1, FDIM), lambda i: (0, 0)),
        ],
        out_specs=pl.BlockSpec((BLK, FDIM), lambda i: (i, 0)),
        out_shape=jax.ShapeDtypeStruct((NPAD, FDIM), jnp.float32),
    )(parts, y, dinv_col, b_gcn2)


def _k6_body(q_ref, r_ref, wcol_ref, cp0_ref, cp1_ref, wl_ref, bl_ref,
             wl2, bl2, wr2, w0, b0, g0, be0, w1, b1, g1, be1,
             w2, b2, g2, be2, w3, b3, out_ref, s_ref):
    i = pl.program_id(0)
    wcol = jnp.reshape(wcol_ref[...], (BLK, 1))
    agg = (q_ref[0] + q_ref[1]) * wcol
    h1 = (jnp.dot(agg, wl_ref[...], preferred_element_type=jnp.float32)
          + bl_ref[...] + r_ref[...])
    h1 = jnp.maximum(h1, 0.0)
    rows = i * BLK + lax.broadcasted_iota(jnp.int32, (BLK, 1), 0)
    h1 = jnp.where(rows < N_NODES, h1, 0.0)
    ccol = jnp.reshape(cp0_ref[...] + cp1_ref[...], (BLK, 1))
    s0 = jnp.sum(h1, axis=0, keepdims=True)
    s1 = jnp.sum(ccol * h1, axis=0, keepdims=True)
    blk = jnp.concatenate([s0, s1], axis=0)

    @pl.when(i == 0)
    def _():
        s_ref[...] = blk

    @pl.when(i > 0)
    def _():
        s_ref[...] += blk

    @pl.when(i == GRID - 1)
    def _():
        inv = 1.0 / jnp.sqrt(jnp.float32(1.0 + 1e-5))
        m_h1 = s_ref[0:1, :] * (1.0 / N_NODES)
        m_agg = s_ref[1:2, :] * (1.0 / N_NODES)
        p = (jnp.dot(m_agg, wl2[...], preferred_element_type=jnp.float32)
             + bl2[...]
             + jnp.dot(m_h1, wr2[...], preferred_element_type=jnp.float32))
        z = jnp.dot(p, w0[...], preferred_element_type=jnp.float32) + b0[...]
        z = jnp.tanh(z * inv * g0[...] + be0[...])
        z = jnp.dot(z, w1[...], preferred_element_type=jnp.float32) + b1[...]
        z = jnp.tanh(z * inv * g1[...] + be1[...])
        z = jnp.dot(z, w2[...], preferred_element_type=jnp.float32) + b2[...]
        z = jnp.tanh(z * inv * g2[...] + be2[...])
        z = jnp.dot(z, w3[...], preferred_element_type=jnp.float32) + b3[...]
        z = z - jnp.max(z, axis=1, keepdims=True)
        ez = jnp.exp(z)
        out_ref[...] = ez / jnp.sum(ez, axis=1, keepdims=True)


def _k6_call(parts2, R, w_col, cp0, cp1, Wl1, bl1_2, *head):
    full = lambda a: pl.BlockSpec(a.shape, lambda i: tuple(0 for _ in a.shape))
    return pl.pallas_call(
        _k6_body,
        grid=(GRID,),
        in_specs=[
            pl.BlockSpec((NC, BLK, FDIM), lambda i: (0, i, 0)),
            pl.BlockSpec((BLK, FDIM), lambda i: (i, 0)),
            pl.BlockSpec((BLK,), lambda i: (i,)),
            pl.BlockSpec((BLK,), lambda i: (i,)),
            pl.BlockSpec((BLK,), lambda i: (i,)),
            pl.BlockSpec((FDIM, FDIM), lambda i: (0, 0)),
            pl.BlockSpec((1, FDIM), lambda i: (0, 0)),
        ] + [full(a) for a in head],
        out_specs=pl.BlockSpec((1, 10), lambda i: (0, 0)),
        out_shape=jax.ShapeDtypeStruct((1, 10), jnp.float32),
        scratch_shapes=[pltpu.VMEM((2, FDIM), jnp.float32)],
    )(parts2, R, w_col, cp0, cp1, Wl1, bl1_2, *head)


def kernel(x, edge_index, W_gcn, b_gcn, Wl1, bl1, Wr1, Wl2, bl2, Wr2,
           W0, b0, g0, be0, W1, b1, g1, be1, W2, b2, g2, be2, W3, b3):
    f32 = jnp.float32
    e = edge_index.shape[1]
    e3 = edge_index.reshape(2, NCH_REAL, ECHUNK)
    tail = lax.slice(edge_index, (0, ((NW - 1) * CPT + 16) * ECHUNK), (2, e))
    pad_np = np.stack([np.arange(PADE) % N_NODES,
                       N_NODES + np.arange(PADE) % TRASH]).astype(np.int32)
    extra3 = jnp.concatenate([tail, jnp.asarray(pad_np)],
                             axis=1).reshape(2, EX, ECHUNK)
    ones_c = jnp.asarray(np.ones((ECHUNK,), np.float32))
    zeros1_c = jnp.asarray(np.zeros((RPT,), np.float32))
    cnt0, cnt1 = _count_call(e3, extra3, ones_c, zeros1_c)
    xw = _mm_call(x, W_gcn)
    y, dinv_col, w_col = _k2_call(xw, cnt0, cnt1)
    parts, cp0, cp1 = _scatter_coef_call(y, e3, extra3, w_col, zeros1_c)
    h0 = _k4_call(parts, y, dinv_col, b_gcn.reshape(1, FDIM))
    parts2 = _scatter_call(h0, e3, extra3)
    R = _mm_call(h0, Wr1)
    return _k6_call(parts2, R, w_col, cp0, cp1,
                    Wl1, bl1.reshape(1, FDIM),
                    Wl2, bl2.reshape(1, FDIM), Wr2,
                    W0, b0.reshape(1, 200), g0.reshape(1, 200),
                    be0.reshape(1, 200),
                    W1, b1.reshape(1, 100), g1.reshape(1, 100),
                    be1.reshape(1, 100),
                    W2, b2.reshape(1, 50), g2.reshape(1, 50),
                    be2.reshape(1, 50),
                    W3, b3.reshape(1, 10))


# speedup vs baseline: 1.0042x; 1.0042x over previous
import functools

import jax
import jax.numpy as jnp
import numpy as np
from jax import lax
from jax.experimental import pallas as pl
from jax.experimental.pallas import tpu as pltpu
from jax.experimental.pallas import tpu_sc as plsc

N_NODES = 10000
FDIM = 128
NC = 2
NS = 16
NW = NC * NS
NPAD = 10240
TRASH = NPAD - N_NODES
ECHUNK = 128
NCH_REAL = 2500
CPT = 80
HC = 40
TAIL = NCH_REAL - (NW - 1) * CPT
EX = CPT - 16
RPT = NPAD // NS
PADE = (EX - 4) * ECHUNK
BLK = 1024
GRID = NPAD // BLK

_mesh = plsc.VectorSubcoreMesh(core_axis_name="c", subcore_axis_name="s")
_sc_params = pltpu.CompilerParams(needs_layout_passes=False)


def _count_body(edge_hbm, extra_hbm, ones_hbm, zeros_hbm, out0, out1,
                dst_v, ones_v, sem, cnt_sh):
    c = lax.axis_index("c")
    s = lax.axis_index("s")
    wid = c * NS + s

    pltpu.sync_copy(zeros_hbm, cnt_sh.at[pl.ds(s * RPT, RPT)])
    pltpu.sync_copy(ones_hbm, ones_v)

    @pl.when(wid < NW - 1)
    def _():
        pltpu.sync_copy(edge_hbm.at[1, pl.ds(wid * CPT, CPT)], dst_v)

    @pl.when(wid == NW - 1)
    def _():
        pltpu.sync_copy(edge_hbm.at[1, pl.ds((NW - 1) * CPT, 16)],
                        dst_v.at[pl.ds(0, 16)])
        pltpu.sync_copy(extra_hbm.at[1], dst_v.at[pl.ds(16, EX)])

    plsc.subcore_barrier()

    def _step(g, carry):
        for u in range(16):
            pltpu.async_copy(ones_v, cnt_sh.at[dst_v.at[g * 16 + u]], sem,
                             add=True)
        for u in range(16):
            pltpu.make_async_copy(ones_v, cnt_sh.at[dst_v.at[g * 16 + u]],
                                  sem).wait()
        return carry

    lax.fori_loop(0, CPT // 16, _step, 0)
    plsc.subcore_barrier()

    @pl.when(c == 0)
    def _():
        pltpu.sync_copy(cnt_sh.at[pl.ds(s * RPT, RPT)],
                        out0.at[pl.ds(s * RPT, RPT)])

    @pl.when(c == 1)
    def _():
        pltpu.sync_copy(cnt_sh.at[pl.ds(s * RPT, RPT)],
                        out1.at[pl.ds(s * RPT, RPT)])


_count_call = pl.kernel(
    _count_body,
    out_type=[jax.ShapeDtypeStruct((NPAD,), jnp.float32),
              jax.ShapeDtypeStruct((NPAD,), jnp.float32)],
    mesh=_mesh,
    scratch_types=[
        pltpu.VMEM((CPT, ECHUNK), jnp.int32),
        pltpu.VMEM((ECHUNK,), jnp.float32),
        pltpu.SemaphoreType.DMA,
        pltpu.VMEM_SHARED((NPAD,), jnp.float32),
    ],
    compiler_params=_sc_params,
)


def _scatter_body(coef, *refs):
    if coef:
        (table_hbm, edge_hbm, extra_hbm, w_hbm, zeros_hbm,
         part_out, cp0_out, cp1_out,
         src_v, dst_v, rows_a, rows_b, sem_a, sem_b,
         wbuf_a, wbuf_b, sem_wa, sem_wb, sem_c, acc_sh, c_sh) = refs
    else:
        (table_hbm, edge_hbm, extra_hbm, part_out,
         src_v, dst_v, rows_a, rows_b, sem_a, sem_b, acc_sh) = refs
    c = lax.axis_index("c")
    s = lax.axis_index("s")
    wid = c * NS + s

    def _zrow(r, carry):
        for k in range(FDIM // 16):
            rows_b[r, pl.ds(k * 16, 16)] = jnp.zeros((16,), jnp.float32)
        return carry

    lax.fori_loop(0, ECHUNK, _zrow, 0)
    for t in range(RPT // ECHUNK):
        pltpu.async_copy(rows_b,
                         acc_sh.at[pl.ds(s * RPT + t * ECHUNK, ECHUNK)],
                         sem_b)
    if coef:
        pltpu.sync_copy(zeros_hbm, c_sh.at[pl.ds(s * RPT, RPT)])

    def _gather(j, rbuf, rsem, wbuf, wsem, climit):
        pltpu.async_copy(table_hbm.at[src_v.at[j]], rbuf, rsem)
        if coef:
            @pl.when(j < climit)
            def _():
                pltpu.async_copy(w_hbm.at[dst_v.at[j]], wbuf, wsem)

    def _drain_scatter(j, rbuf, rsem, wbuf, wsem, climit):
        pltpu.make_async_copy(table_hbm.at[src_v.at[j]], rbuf, rsem).wait()
        if coef:
            @pl.when(j < climit)
            def _():
                pltpu.make_async_copy(w_hbm.at[dst_v.at[j]], wbuf,
                                      wsem).wait()
                pltpu.async_copy(wbuf, c_sh.at[src_v.at[j]], sem_c, add=True)
        pltpu.sync_copy(rbuf, acc_sh.at[dst_v.at[j]], add=True)
        if coef:
            @pl.when(j < climit)
            def _():
                pltpu.make_async_copy(wbuf, c_sh.at[src_v.at[j]],
                                      sem_c).wait()

    def _pipeline(climit):
        _gather(0, rows_a, sem_a, wbuf_a if coef else None,
                sem_wa if coef else None, climit)

        def _step(j2, carry):
            j = j2 * 2
            _gather(j + 1, rows_b, sem_b, wbuf_b if coef else None,
                    sem_wb if coef else None, climit)
            _drain_scatter(j, rows_a, sem_a, wbuf_a if coef else None,
                           sem_wa if coef else None, climit)

            @pl.when(j + 2 < HC)
            def _():
                _gather(j + 2, rows_a, sem_a, wbuf_a if coef else None,
                        sem_wa if coef else None, climit)

            _drain_scatter(j + 1, rows_b, sem_b, wbuf_b if coef else None,
                           sem_wb if coef else None, climit)
            return carry

        lax.fori_loop(0, HC // 2, _step, 0)

    last = wid == NW - 1
    @pl.when(jnp.logical_not(last))
    def _():
        pltpu.sync_copy(edge_hbm.at[0, pl.ds(wid * CPT, HC)], src_v)
        pltpu.sync_copy(edge_hbm.at[1, pl.ds(wid * CPT, HC)], dst_v)

    @pl.when(last)
    def _():
        pltpu.sync_copy(edge_hbm.at[0, pl.ds((NW - 1) * CPT, 16)],
                        src_v.at[pl.ds(0, 16)])
        pltpu.sync_copy(edge_hbm.at[1, pl.ds((NW - 1) * CPT, 16)],
                        dst_v.at[pl.ds(0, 16)])
        pltpu.sync_copy(extra_hbm.at[0, pl.ds(0, HC - 16)],
                        src_v.at[pl.ds(16, HC - 16)])
        pltpu.sync_copy(extra_hbm.at[1, pl.ds(0, HC - 16)],
                        dst_v.at[pl.ds(16, HC - 16)])

    for t in range(RPT // ECHUNK):
        pltpu.make_async_copy(
            rows_b, acc_sh.at[pl.ds(s * RPT + t * ECHUNK, ECHUNK)],
            sem_b).wait()
    plsc.subcore_barrier()
    _pipeline(jnp.where(last, 20, HC))
    @pl.when(jnp.logical_not(last))
    def _():
        pltpu.sync_copy(edge_hbm.at[0, pl.ds(wid * CPT + HC, HC)], src_v)
        pltpu.sync_copy(edge_hbm.at[1, pl.ds(wid * CPT + HC, HC)], dst_v)

    @pl.when(last)
    def _():
        pltpu.sync_copy(extra_hbm.at[0, pl.ds(HC - 16, HC)], src_v)
        pltpu.sync_copy(extra_hbm.at[1, pl.ds(HC - 16, HC)], dst_v)

    _pipeline(jnp.where(last, 0, HC))

    plsc.subcore_barrier()
    pltpu.sync_copy(acc_sh.at[pl.ds(s * RPT, RPT)],
                    part_out.at[c, pl.ds(s * RPT, RPT)])
    if coef:
        @pl.when(c == 0)
        def _():
            pltpu.sync_copy(c_sh.at[pl.ds(s * RPT, RPT)],
                            cp0_out.at[pl.ds(s * RPT, RPT)])

        @pl.when(c == 1)
        def _():
            pltpu.sync_copy(c_sh.at[pl.ds(s * RPT, RPT)],
                            cp1_out.at[pl.ds(s * RPT, RPT)])


def _make_scatter(coef):
    part_type = jax.ShapeDtypeStruct((NC, NPAD, FDIM), jnp.float32)
    out_type = [part_type]
    scratch = [
        pltpu.VMEM((HC, ECHUNK), jnp.int32),
        pltpu.VMEM((HC, ECHUNK), jnp.int32),
        pltpu.VMEM((ECHUNK, FDIM), jnp.float32),
        pltpu.VMEM((ECHUNK, FDIM), jnp.float32),
        pltpu.SemaphoreType.DMA,
        pltpu.SemaphoreType.DMA,
    ]
    if coef:
        out_type += [jax.ShapeDtypeStruct((NPAD,), jnp.float32),
                     jax.ShapeDtypeStruct((NPAD,), jnp.float32)]
        scratch += [
            pltpu.VMEM((ECHUNK,), jnp.float32),
            pltpu.VMEM((ECHUNK,), jnp.float32),
            pltpu.SemaphoreType.DMA,
            pltpu.SemaphoreType.DMA,
            pltpu.SemaphoreType.DMA,
        ]
    scratch.append(pltpu.VMEM_SHARED((NPAD, FDIM), jnp.float32))
    if coef:
        scratch.append(pltpu.VMEM_SHARED((NPAD,), jnp.float32))
    return pl.kernel(
        functools.partial(_scatter_body, coef),
        out_type=out_type if coef else part_type,
        mesh=_mesh,
        scratch_types=scratch,
        compiler_params=_sc_params,
    )


_scatter_coef_call = _make_scatter(True)
_scatter_call = _make_scatter(False)


def _mm_body(x_ref, w_ref, o_ref):
    o_ref[...] = jnp.dot(x_ref[...], w_ref[...],
                         preferred_element_type=jnp.float32)


def _mm_call(x, W):
    nblk = x.shape[0] // GRID
    return pl.pallas_call(
        _mm_body,
        grid=(GRID,),
        in_specs=[
            pl.BlockSpec((nblk, FDIM), lambda i: (i, 0)),
            pl.BlockSpec((FDIM, FDIM), lambda i: (0, 0)),
        ],
        out_specs=pl.BlockSpec((nblk, FDIM), lambda i: (i, 0)),
        out_shape=jax.ShapeDtypeStruct((NPAD, FDIM), jnp.float32),
    )(x, W)


def _k2_body(xw_ref, c0_ref, c1_ref, y_ref, dinv_ref, wcol_ref):
    cnt = c0_ref[...] + c1_ref[...]
    dinv = lax.rsqrt(cnt + 1.0)
    y_ref[...] = xw_ref[...] * jnp.reshape(dinv, (BLK, 1))
    dinv_ref[...] = dinv
    wcol_ref[...] = 1.0 / jnp.maximum(cnt, 1.0)


def _k2_call(xw, cnt0, cnt1):
    return pl.pallas_call(
        _k2_body,
        grid=(GRID,),
        in_specs=[
            pl.BlockSpec((BLK, FDIM), lambda i: (i, 0)),
            pl.BlockSpec((BLK,), lambda i: (i,)),
            pl.BlockSpec((BLK,), lambda i: (i,)),
        ],
        out_specs=[
            pl.BlockSpec((BLK, FDIM), lambda i: (i, 0)),
            pl.BlockSpec((BLK,), lambda i: (i,)),
            pl.BlockSpec((BLK,), lambda i: (i,)),
        ],
        out_shape=[
            jax.ShapeDtypeStruct((NPAD, FDIM), jnp.float32),
            jax.ShapeDtypeStruct((NPAD,), jnp.float32),
            jax.ShapeDtypeStruct((NPAD,), jnp.float32),
        ],
    )(xw, cnt0, cnt1)


def _k4_body(p_ref, y_ref, dinv_ref, b_ref, h0_ref):
    dcol = jnp.reshape(dinv_ref[...], (BLK, 1))
    t = (p_ref[0] + p_ref[1] + y_ref[...]) * dcol
    h0_ref[...] = jnp.maximum(t + b_ref[...], 0.0)


def _k4_call(parts, y, dinv_col, b_gcn2):
    return pl.pallas_call(
        _k4_body,
        grid=(GRID,),
        in_specs=[
            pl.BlockSpec((NC, BLK, FDIM), lambda i: (0, i, 0)),
            pl.BlockSpec((BLK, FDIM), lambda i: (i, 0)),
            pl.BlockSpec((BLK,), lambda i: (i,)),
            pl.BlockSpec((1, FDIM), lambda i: (0, 0)),
        ],
        out_specs=pl.BlockSpec((BLK, FDIM), lambda i: (i, 0)),
        out_shape=jax.ShapeDtypeStruct((NPAD, FDIM), jnp.float32),
    )(parts, y, dinv_col, b_gcn2)


def _k6_body(q_ref, r_ref, wcol_ref, cp0_ref, cp1_ref, wl_ref, bl_ref,
             wl2, bl2, wr2, w0, b0, g0, be0, w1, b1, g1, be1,
             w2, b2, g2, be2, w3, b3, out_ref, s_ref):
    i = pl.program_id(0)
    wcol = jnp.reshape(wcol_ref[...], (BLK, 1))
    agg = (q_ref[0] + q_ref[1]) * wcol
    h1 = (jnp.dot(agg, wl_ref[...], preferred_element_type=jnp.float32)
          + bl_ref[...] + r_ref[...])
    h1 = jnp.maximum(h1, 0.0)
    rows = i * BLK + lax.broadcasted_iota(jnp.int32, (BLK, 1), 0)
    h1 = jnp.where(rows < N_NODES, h1, 0.0)
    ccol = jnp.reshape(cp0_ref[...] + cp1_ref[...], (BLK, 1))
    s0 = jnp.sum(h1, axis=0, keepdims=True)
    s1 = jnp.sum(ccol * h1, axis=0, keepdims=True)
    blk = jnp.concatenate([s0, s1], axis=0)

    @pl.when(i == 0)
    def _():
        s_ref[...] = blk

    @pl.when(i > 0)
    def _():
        s_ref[...] += blk

    @pl.when(i == GRID - 1)
    def _():
        inv = 1.0 / jnp.sqrt(jnp.float32(1.0 + 1e-5))
        m_h1 = s_ref[0:1, :] * (1.0 / N_NODES)
        m_agg = s_ref[1:2, :] * (1.0 / N_NODES)
        p = (jnp.dot(m_agg, wl2[...], preferred_element_type=jnp.float32)
             + bl2[...]
             + jnp.dot(m_h1, wr2[...], preferred_element_type=jnp.float32))
        z = jnp.dot(p, w0[...], preferred_element_type=jnp.float32) + b0[...]
        z = jnp.tanh(z * inv * g0[...] + be0[...])
        z = jnp.dot(z, w1[...], preferred_element_type=jnp.float32) + b1[...]
        z = jnp.tanh(z * inv * g1[...] + be1[...])
        z = jnp.dot(z, w2[...], preferred_element_type=jnp.float32) + b2[...]
        z = jnp.tanh(z * inv * g2[...] + be2[...])
        z = jnp.dot(z, w3[...], preferred_element_type=jnp.float32) + b3[...]
        z = z - jnp.max(z, axis=1, keepdims=True)
        ez = jnp.exp(z)
        out_ref[...] = ez / jnp.sum(ez, axis=1, keepdims=True)


def _k6_call(parts2, R, w_col, cp0, cp1, Wl1, bl1_2, *head):
    full = lambda a: pl.BlockSpec(a.shape, lambda i: tuple(0 for _ in a.shape))
    return pl.pallas_call(
        _k6_body,
        grid=(GRID,),
        in_specs=[
            pl.BlockSpec((NC, BLK, FDIM), lambda i: (0, i, 0)),
            pl.BlockSpec((BLK, FDIM), lambda i: (i, 0)),
            pl.BlockSpec((BLK,), lambda i: (i,)),
            pl.BlockSpec((BLK,), lambda i: (i,)),
            pl.BlockSpec((BLK,), lambda i: (i,)),
            pl.BlockSpec((FDIM, FDIM), lambda i: (0, 0)),
            pl.BlockSpec((1, FDIM), lambda i: (0, 0)),
        ] + [full(a) for a in head],
        out_specs=pl.BlockSpec((1, 10), lambda i: (0, 0)),
        out_shape=jax.ShapeDtypeStruct((1, 10), jnp.float32),
        scratch_shapes=[pltpu.VMEM((2, FDIM), jnp.float32)],
    )(parts2, R, w_col, cp0, cp1, Wl1, bl1_2, *head)


def kernel(x, edge_index, W_gcn, b_gcn, Wl1, bl1, Wr1, Wl2, bl2, Wr2,
           W0, b0, g0, be0, W1, b1, g1, be1, W2, b2, g2, be2, W3, b3):
    f32 = jnp.float32
    e = edge_index.shape[1]
    e3 = edge_index.reshape(2, NCH_REAL, ECHUNK)
    tail = lax.slice(edge_index, (0, ((NW - 1) * CPT + 16) * ECHUNK), (2, e))
    pad_np = np.stack([np.arange(PADE) % N_NODES,
                       N_NODES + np.arange(PADE) % TRASH]).astype(np.int32)
    extra3 = jnp.concatenate([tail, jnp.asarray(pad_np)],
                             axis=1).reshape(2, EX, ECHUNK)
    ones_c = jnp.asarray(np.ones((ECHUNK,), np.float32))
    zeros1_c = jnp.asarray(np.zeros((RPT,), np.float32))
    cnt0, cnt1 = _count_call(e3, extra3, ones_c, zeros1_c)
    xw = _mm_call(x, W_gcn)
    y, dinv_col, w_col = _k2_call(xw, cnt0, cnt1)
    parts, cp0, cp1 = _scatter_coef_call(y, e3, extra3, w_col, zeros1_c)
    h0 = _k4_call(parts, y, dinv_col, b_gcn.reshape(1, FDIM))
    parts2 = _scatter_call(h0, e3, extra3)
    R = _mm_call(h0, Wr1)
    return _k6_call(parts2, R, w_col, cp0, cp1,
                    Wl1, bl1.reshape(1, FDIM),
                    Wl2, bl2.reshape(1, FDIM), Wr2,
                    W0, b0.reshape(1, 200), g0.reshape(1, 200),
                    be0.reshape(1, 200),
                    W1, b1.reshape(1, 100), g1.reshape(1, 100),
                    be1.reshape(1, 100),
                    W2, b2.reshape(1, 50), g2.reshape(1, 50),
                    be2.reshape(1, 50),
                    W3, b3.reshape(1, 10))
